# baseline (speedup 1.0000x reference)
import functools

import jax
import jax.numpy as jnp
from jax import lax
from jax.experimental import pallas as pl
from jax.experimental.pallas import tpu as pltpu
from jax.experimental.pallas import tpu_sc as plsc

_U_DIM = 100000
_Z_DIM = 64
_BATCH = 16384

_NC = 2
_NS = 16
_NW = _NC * _NS
_BPW = _BATCH // _NW

_mesh = plsc.VectorSubcoreMesh(core_axis_name="c", subcore_axis_name="s")


@functools.partial(
    pl.kernel,
    mesh=_mesh,
    compiler_params=pltpu.CompilerParams(use_tc_tiling_on_sc=False),
    out_type=(
        jax.ShapeDtypeStruct((_BATCH, _Z_DIM), jnp.float32),
        jax.ShapeDtypeStruct((_BATCH, _Z_DIM), jnp.float32),
    ),
    scratch_types=[
        pltpu.VMEM((_BPW,), jnp.int32),
        pltpu.VMEM((_BPW, _Z_DIM), jnp.float32),
        pltpu.VMEM((_BPW, _Z_DIM), jnp.float32),
        pltpu.SemaphoreType.DMA,
        pltpu.SemaphoreType.DMA,
    ],
)
def _dual_gather(mean_hbm, logvar_hbm, idx_hbm, out_mean, out_logvar,
                 idx_v, rows_m, rows_lv, sem_m, sem_lv):
    wid = lax.axis_index("s") * _NC + lax.axis_index("c")
    base = wid * _BPW
    pltpu.sync_copy(idx_hbm.at[pl.ds(base, _BPW)], idx_v)
    cm = pltpu.async_copy(mean_hbm.at[idx_v], rows_m, sem_m)
    clv = pltpu.async_copy(logvar_hbm.at[idx_v], rows_lv, sem_lv)
    cm.wait()
    pltpu.sync_copy(rows_m, out_mean.at[pl.ds(base, _BPW)])
    clv.wait()
    pltpu.sync_copy(rows_lv, out_logvar.at[pl.ds(base, _BPW)])


def kernel(u, embed_mean, embed_log_var):
    return _dual_gather(embed_mean, embed_log_var, u.astype(jnp.int32))


# speedup vs baseline: 2.0057x; 2.0057x over previous
import functools

import jax
import jax.numpy as jnp
from jax import lax
from jax.experimental import pallas as pl
from jax.experimental.pallas import tpu as pltpu
from jax.experimental.pallas import tpu_sc as plsc

_U_DIM = 100000
_Z_DIM = 64
_BATCH = 16384

_NC = 2
_NS = 16
_NW = _NC * _NS
_CHUNK = 4096

_mesh = plsc.VectorSubcoreMesh(core_axis_name="c", subcore_axis_name="s")


@functools.partial(
    pl.kernel,
    mesh=_mesh,
    compiler_params=pltpu.CompilerParams(needs_layout_passes=False),
    out_type=(
        jax.ShapeDtypeStruct((_Z_DIM, _BATCH), jnp.float32),
        jax.ShapeDtypeStruct((_Z_DIM, _BATCH), jnp.float32),
    ),
    scratch_types=[
        pltpu.VMEM((_BATCH,), jnp.int32),
        pltpu.VMEM((_U_DIM,), jnp.float32),
        pltpu.VMEM((_CHUNK,), jnp.float32),
    ],
)
def _tgather(mt, lt, u_hbm, om, ol, u_v, row_v, out_v):
    wid = lax.axis_index("s") * _NC + lax.axis_index("c")
    pltpu.sync_copy(u_hbm, u_v)
    for k in range(2):
        d = wid * 2 + k
        for src, dst in ((mt, om), (lt, ol)):
            pltpu.sync_copy(src.at[d], row_v)
            for c in range(_BATCH // _CHUNK):

                def body(i, carry):
                    idx = u_v[pl.ds(c * _CHUNK + i * 16, 16)]
                    out_v[pl.ds(i * 16, 16)] = plsc.load_gather(row_v, [idx])
                    return carry

                lax.fori_loop(0, _CHUNK // 16, body, 0)
                pltpu.sync_copy(out_v, dst.at[d, pl.ds(c * _CHUNK, _CHUNK)])


def kernel(u, embed_mean, embed_log_var):
    om, ol = _tgather(embed_mean.T, embed_log_var.T, u.astype(jnp.int32))
    return om.T, ol.T


# speedup vs baseline: 2.4262x; 1.2096x over previous
import functools

import jax
import jax.numpy as jnp
from jax import lax
from jax.experimental import pallas as pl
from jax.experimental.pallas import tpu as pltpu
from jax.experimental.pallas import tpu_sc as plsc

_U_DIM = 100000
_Z_DIM = 64
_BATCH = 16384

_NC = 2
_NS = 16
_NW = _NC * _NS
_CHUNK = 4096

_mesh = plsc.VectorSubcoreMesh(core_axis_name="c", subcore_axis_name="s")


@functools.partial(
    pl.kernel,
    mesh=_mesh,
    compiler_params=pltpu.CompilerParams(needs_layout_passes=False),
    out_type=(
        jax.ShapeDtypeStruct((_Z_DIM, _BATCH), jnp.float32),
        jax.ShapeDtypeStruct((_Z_DIM, _BATCH), jnp.float32),
    ),
    scratch_types=[
        pltpu.VMEM((_BATCH,), jnp.int32),
        pltpu.VMEM((_U_DIM,), jnp.float32),
        pltpu.VMEM((_CHUNK,), jnp.float32),
    ],
)
def _tgather(mt, lt, u_hbm, om, ol, u_v, row_v, out_v):
    wid = lax.axis_index("s") * _NC + lax.axis_index("c")
    pltpu.sync_copy(u_hbm, u_v)
    unroll = 16
    for k in range(2):
        d = wid * 2 + k
        for src, dst in ((mt, om), (lt, ol)):
            pltpu.sync_copy(src.at[d], row_v)
            for c in range(_BATCH // _CHUNK):

                def body(i, carry):
                    base = i * (16 * unroll)
                    for j in range(unroll):
                        idx = u_v[pl.ds(c * _CHUNK + base + j * 16, 16)]
                        out_v[pl.ds(base + j * 16, 16)] = plsc.load_gather(
                            row_v, [idx]
                        )
                    return carry

                lax.fori_loop(0, _CHUNK // (16 * unroll), body, 0)
                pltpu.sync_copy(out_v, dst.at[d, pl.ds(c * _CHUNK, _CHUNK)])


def kernel(u, embed_mean, embed_log_var):
    om, ol = _tgather(embed_mean.T, embed_log_var.T, u.astype(jnp.int32))
    return om.T, ol.T


# speedup vs baseline: 2.5262x; 1.0412x over previous
import functools

import jax
import jax.numpy as jnp
from jax import lax
from jax.experimental import pallas as pl
from jax.experimental.pallas import tpu as pltpu
from jax.experimental.pallas import tpu_sc as plsc

_U_DIM = 100000
_Z_DIM = 64
_BATCH = 16384

_NC = 2
_NS = 16
_NW = _NC * _NS
_CHUNK = 4096

_mesh = plsc.VectorSubcoreMesh(core_axis_name="c", subcore_axis_name="s")


@functools.partial(
    pl.kernel,
    mesh=_mesh,
    compiler_params=pltpu.CompilerParams(needs_layout_passes=False),
    out_type=(
        jax.ShapeDtypeStruct((_Z_DIM, _BATCH), jnp.float32),
        jax.ShapeDtypeStruct((_Z_DIM, _BATCH), jnp.float32),
    ),
    scratch_types=[
        pltpu.VMEM((_BATCH,), jnp.int32),
        pltpu.VMEM((_U_DIM,), jnp.float32),
        pltpu.VMEM((_CHUNK,), jnp.float32),
    ],
)
def _tgather(mt, lt, u_hbm, om, ol, u_v, row_v, out_v):
    wid = lax.axis_index("s") * _NC + lax.axis_index("c")
    pltpu.sync_copy(u_hbm, u_v)
    unroll = 16
    for k in range(2):
        d = wid * 2 + k
        for src, dst in ((mt, om), (lt, ol)):
            pltpu.sync_copy(src.at[d], row_v)
            for c in range(_BATCH // _CHUNK):

                @plsc.parallel_loop(0, _CHUNK, step=16 * unroll)
                def body(i):
                    for j in range(unroll):
                        idx = u_v[pl.ds(c * _CHUNK + i + j * 16, 16)]
                        out_v[pl.ds(i + j * 16, 16)] = plsc.load_gather(
                            row_v, [idx]
                        )

                pltpu.sync_copy(out_v, dst.at[d, pl.ds(c * _CHUNK, _CHUNK)])


def kernel(u, embed_mean, embed_log_var):
    om, ol = _tgather(embed_mean.T, embed_log_var.T, u.astype(jnp.int32))
    return om.T, ol.T


# speedup vs baseline: 2.5398x; 1.0054x over previous
import functools

import jax
import jax.numpy as jnp
from jax import lax
from jax.experimental import pallas as pl
from jax.experimental.pallas import tpu as pltpu
from jax.experimental.pallas import tpu_sc as plsc

_U_DIM = 100000
_Z_DIM = 64
_BATCH = 16384

_NC = 2
_NS = 16
_NW = _NC * _NS
_CHUNK = 4096
_UNROLL = 16

_mesh = plsc.VectorSubcoreMesh(core_axis_name="c", subcore_axis_name="s")


@functools.partial(
    pl.kernel,
    mesh=_mesh,
    compiler_params=pltpu.CompilerParams(needs_layout_passes=False),
    out_type=(
        jax.ShapeDtypeStruct((_Z_DIM, _BATCH), jnp.float32),
        jax.ShapeDtypeStruct((_Z_DIM, _BATCH), jnp.float32),
    ),
    scratch_types=[
        pltpu.VMEM((_BATCH,), jnp.int32),
        pltpu.VMEM((_U_DIM,), jnp.float32),
        pltpu.VMEM((2, _CHUNK), jnp.float32),
        pltpu.SemaphoreType.DMA,
        pltpu.SemaphoreType.DMA,
    ],
)
def _tgather(mt, lt, u_hbm, om, ol, u_v, row_v, out_v, osem0, osem1):
    wid = lax.axis_index("s") * _NC + lax.axis_index("c")
    pltpu.sync_copy(u_hbm, u_v)
    osems = (osem0, osem1)
    pending = [None, None]
    for k in range(2):
        d = wid * 2 + k
        for src, dst in ((mt, om), (lt, ol)):
            pltpu.sync_copy(src.at[d], row_v)
            for c in range(_BATCH // _CHUNK):
                buf = c % 2
                if pending[buf] is not None:
                    pending[buf].wait()
                    pending[buf] = None

                @plsc.parallel_loop(0, _CHUNK, step=16 * _UNROLL)
                def body(i):
                    for j in range(_UNROLL):
                        idx = u_v[pl.ds(c * _CHUNK + i + j * 16, 16)]
                        out_v[buf, pl.ds(i + j * 16, 16)] = plsc.load_gather(
                            row_v, [idx]
                        )

                pending[buf] = pltpu.async_copy(
                    out_v.at[buf], dst.at[d, pl.ds(c * _CHUNK, _CHUNK)],
                    osems[buf],
                )
    for buf in range(2):
        if pending[buf] is not None:
            pending[buf].wait()


def kernel(u, embed_mean, embed_log_var):
    om, ol = _tgather(embed_mean.T, embed_log_var.T, u.astype(jnp.int32))
    return om.T, ol.T
